# Initial kernel scaffold; baseline (speedup 1.0000x reference)
import jax
import jax.numpy as jnp
from jax.experimental import pallas as pl


def kernel(x, edge_attr, params, edge_index, batch):
    raise NotImplementedError("write your pallas kernel here")


# speedup vs baseline: 2.9455x; 2.9455x over previous
import functools

import jax
import jax.numpy as jnp
from jax import lax
from jax.experimental import pallas as pl
from jax.experimental.pallas import tpu as pltpu
from jax.experimental.pallas import tpu_sc as plsc

ND = 10000
NE = 320000
D = 128
D2 = 256
NG = 128
NL = 5
FC1 = 512
TASKS = 128

BLK = 2000
NB = ND // BLK
EB = 4000
NEB = NE // EB

NTILES = 32
EPT = NE // NTILES
CH = 80
NCH = EPT // CH
NDP = 10240
RPT = NDP // 16


def _sc_msg_body(hcur_hbm, e_hbm, src_hbm, dst_hbm, zeros_hbm, out_hbm,
                 src_v, dst_v, rows_v, e_v, agg_sh, sem):
    cid = lax.axis_index("c")
    sid = lax.axis_index("s")
    wid = cid * 16 + sid
    pltpu.sync_copy(zeros_hbm.at[pl.ds(sid * RPT, RPT)],
                    agg_sh.at[pl.ds(sid * RPT, RPT)])
    plsc.subcore_barrier()
    ebase = wid * EPT

    def chunk(c, carry):
        base = ebase + c * CH
        pltpu.sync_copy(src_hbm.at[pl.ds(base, CH)], src_v)
        pltpu.sync_copy(dst_hbm.at[pl.ds(base, CH)], dst_v)
        gat = pltpu.async_copy(hcur_hbm.at[src_v], rows_v, sem)
        pltpu.sync_copy(e_hbm.at[pl.ds(base, CH)], e_v)
        gat.wait()

        def row(r, carry2):
            for k in range(8):
                sl = pl.ds(k * 16, 16)
                v = rows_v[r, sl] + e_v[r, sl]
                rows_v[r, sl] = jnp.maximum(v, 0.0)
            return carry2

        lax.fori_loop(0, CH, row, 0)
        pltpu.sync_copy(rows_v, agg_sh.at[dst_v], add=True)
        return carry

    lax.fori_loop(0, NCH, chunk, 0)
    plsc.subcore_barrier()
    pltpu.sync_copy(agg_sh.at[pl.ds(sid * RPT, RPT)],
                    out_hbm.at[cid, pl.ds(sid * RPT, RPT)])


_sc_msg = pl.kernel(
    _sc_msg_body,
    out_type=jax.ShapeDtypeStruct((2, NDP, D), jnp.float32),
    mesh=plsc.VectorSubcoreMesh(core_axis_name="c", subcore_axis_name="s"),
    scratch_types=[
        pltpu.VMEM((CH,), jnp.int32),
        pltpu.VMEM((CH,), jnp.int32),
        pltpu.VMEM((CH, D), jnp.float32),
        pltpu.VMEM((CH, D), jnp.float32),
        pltpu.VMEM_SHARED((NDP, D), jnp.float32),
        pltpu.SemaphoreType.DMA,
    ],
)


def _prep_body(x_ref, w_ref, b_ref, vn0_ref, batch_ref,
               h_ref, hcur_ref, oh_ref):
    h = jnp.dot(x_ref[...], w_ref[...],
                preferred_element_type=jnp.float32) + b_ref[...]
    h_ref[...] = h
    hcur_ref[...] = h + vn0_ref[...]
    gi = lax.broadcasted_iota(jnp.int32, (BLK, NG), 1)
    oh_ref[...] = (batch_ref[...] == gi).astype(jnp.float32)


_prep = pl.pallas_call(
    _prep_body,
    grid=(NB,),
    in_specs=[
        pl.BlockSpec((BLK, D), lambda i: (i, 0)),
        pl.BlockSpec((D, D), lambda i: (0, 0)),
        pl.BlockSpec((1, D), lambda i: (0, 0)),
        pl.BlockSpec((1, D), lambda i: (0, 0)),
        pl.BlockSpec((BLK, 1), lambda i: (i, 0)),
    ],
    out_specs=[
        pl.BlockSpec((BLK, D), lambda i: (i, 0)),
        pl.BlockSpec((BLK, D), lambda i: (i, 0)),
        pl.BlockSpec((BLK, NG), lambda i: (i, 0)),
    ],
    out_shape=[
        jax.ShapeDtypeStruct((ND, D), jnp.float32),
        jax.ShapeDtypeStruct((ND, D), jnp.float32),
        jax.ShapeDtypeStruct((ND, NG), jnp.float32),
    ],
)


def _edge_body(ea_ref, w_ref, b_ref, e_ref):
    e_ref[...] = jnp.dot(ea_ref[...], w_ref[...],
                         preferred_element_type=jnp.float32) + b_ref[...]


_edge_enc = pl.pallas_call(
    _edge_body,
    grid=(NEB,),
    in_specs=[
        pl.BlockSpec((EB, 16), lambda i: (i, 0)),
        pl.BlockSpec((16, D), lambda i: (0, 0)),
        pl.BlockSpec((1, D), lambda i: (0, 0)),
    ],
    out_specs=pl.BlockSpec((EB, D), lambda i: (i, 0)),
    out_shape=jax.ShapeDtypeStruct((NE, D), jnp.float32),
)


def _mlp_body(hcur_ref, agg_ref, sc_ref, w1_ref, b1_ref, w2_ref, b2_ref,
              z1_ref, st_ref):
    i = pl.program_id(0)
    z0 = sc_ref[...] * hcur_ref[...] + agg_ref[0] + agg_ref[1]
    t = jnp.maximum(jnp.dot(z0, w1_ref[...],
                            preferred_element_type=jnp.float32) + b1_ref[...],
                    0.0)
    z1 = jnp.dot(t, w2_ref[...],
                 preferred_element_type=jnp.float32) + b2_ref[...]
    z1_ref[...] = z1
    s1 = jnp.sum(z1, axis=0, keepdims=True)
    st = jnp.concatenate([s1, jnp.zeros((7, D), jnp.float32)], axis=0)

    @pl.when(i == 0)
    def _():
        st_ref[...] = jnp.zeros_like(st_ref)

    st_ref[...] += st


_mlp = pl.pallas_call(
    _mlp_body,
    grid=(NB,),
    in_specs=[
        pl.BlockSpec((BLK, D), lambda i: (i, 0)),
        pl.BlockSpec((2, BLK, D), lambda i: (0, i, 0)),
        pl.BlockSpec((1, D), lambda i: (0, 0)),
        pl.BlockSpec((D, D2), lambda i: (0, 0)),
        pl.BlockSpec((1, D2), lambda i: (0, 0)),
        pl.BlockSpec((D2, D), lambda i: (0, 0)),
        pl.BlockSpec((1, D), lambda i: (0, 0)),
    ],
    out_specs=[
        pl.BlockSpec((BLK, D), lambda i: (i, 0)),
        pl.BlockSpec((8, D), lambda i: (0, 0)),
    ],
    out_shape=[
        jax.ShapeDtypeStruct((ND, D), jnp.float32),
        jax.ShapeDtypeStruct((8, D), jnp.float32),
    ],
)


def _post_body(z1_ref, st_ref, hprev_ref, jk_ref, oh_ref, g_ref, bta_ref,
               vn_ref, vw1_ref, vb1_ref, vw2_ref, vb2_ref,
               z_ref, jk_out_ref, vnnew_ref, ssq_ref, seg_ref, *, last):
    p = pl.program_id(0)
    i = pl.program_id(1)
    mu = st_ref[0:1, :] * (1.0 / ND)

    @pl.when(p == 0)
    def _():
        @pl.when(i == 0)
        def _():
            ssq_ref[...] = jnp.zeros_like(ssq_ref)
            seg_ref[...] = jnp.zeros_like(seg_ref)
            vnnew_ref[...] = jnp.zeros_like(vnnew_ref)

        dz = z1_ref[...] - mu
        ssq_ref[0:1, :] += jnp.sum(dz * dz, axis=0, keepdims=True)

    @pl.when(p == 1)
    def _():
        var = ssq_ref[0:1, :] * (1.0 / ND)
        zn = (g_ref[...] * (z1_ref[...] - mu) / jnp.sqrt(var + 1e-5)
              + bta_ref[...])
        if not last:
            zn = jnp.maximum(zn, 0.0)
        z = zn + hprev_ref[...]
        z_ref[...] = z
        jk_out_ref[...] = jk_ref[...] + z
        seg = lax.dot_general(oh_ref[...], z, (((0,), (0,)), ((), ())),
                              preferred_element_type=jnp.float32,
                              precision=lax.Precision.HIGHEST)
        seg_ref[...] += seg

        @pl.when(i == NB - 1)
        def _():
            vt = seg_ref[...] + vn_ref[...]
            t = jnp.maximum(
                jnp.dot(vt, vw1_ref[...],
                        preferred_element_type=jnp.float32) + vb1_ref[...],
                0.0)
            vnnew_ref[...] = vn_ref[...] + jnp.maximum(
                jnp.dot(t, vw2_ref[...],
                        preferred_element_type=jnp.float32) + vb2_ref[...],
                0.0)


def _make_post(last):
    return pl.pallas_call(
        functools.partial(_post_body, last=last),
        grid=(2, NB),
        in_specs=[
            pl.BlockSpec((BLK, D), lambda p, i: (i, 0)),
            pl.BlockSpec((8, D), lambda p, i: (0, 0)),
            pl.BlockSpec((BLK, D), lambda p, i: (i, 0)),
            pl.BlockSpec((BLK, D), lambda p, i: (i, 0)),
            pl.BlockSpec((BLK, NG), lambda p, i: (i, 0)),
            pl.BlockSpec((1, D), lambda p, i: (0, 0)),
            pl.BlockSpec((1, D), lambda p, i: (0, 0)),
            pl.BlockSpec((NG, D), lambda p, i: (0, 0)),
            pl.BlockSpec((D, D2), lambda p, i: (0, 0)),
            pl.BlockSpec((1, D2), lambda p, i: (0, 0)),
            pl.BlockSpec((D2, D), lambda p, i: (0, 0)),
            pl.BlockSpec((1, D), lambda p, i: (0, 0)),
        ],
        out_specs=[
            pl.BlockSpec((BLK, D), lambda p, i: (i, 0)),
            pl.BlockSpec((BLK, D), lambda p, i: (i, 0)),
            pl.BlockSpec((NG, D), lambda p, i: (0, 0)),
        ],
        out_shape=[
            jax.ShapeDtypeStruct((ND, D), jnp.float32),
            jax.ShapeDtypeStruct((ND, D), jnp.float32),
            jax.ShapeDtypeStruct((NG, D), jnp.float32),
        ],
        scratch_shapes=[
            pltpu.VMEM((8, D), jnp.float32),
            pltpu.VMEM((NG, D), jnp.float32),
        ],
    )


_post_mid = _make_post(last=False)
_post_last = _make_post(last=True)


def _next_body(z_ref, oh_ref, vn_ref, hcur_ref):
    hcur_ref[...] = z_ref[...] + jnp.dot(
        oh_ref[...], vn_ref[...], preferred_element_type=jnp.float32,
        precision=lax.Precision.HIGHEST)


_next_hcur = pl.pallas_call(
    _next_body,
    grid=(NB,),
    in_specs=[
        pl.BlockSpec((BLK, D), lambda i: (i, 0)),
        pl.BlockSpec((BLK, NG), lambda i: (i, 0)),
        pl.BlockSpec((NG, D), lambda i: (0, 0)),
    ],
    out_specs=pl.BlockSpec((BLK, D), lambda i: (i, 0)),
    out_shape=jax.ShapeDtypeStruct((ND, D), jnp.float32),
)


def _head_body(jk_ref, w1_ref, b1_ref, w2_ref, b2_ref, out_ref):
    t = jnp.dot(jk_ref[...], w1_ref[...],
                preferred_element_type=jnp.float32,
                            precision=lax.Precision.HIGHEST) + b1_ref[...]
    t = jnp.where(t > 0, t, 0.1 * t)
    out_ref[...] = jnp.dot(t, w2_ref[...],
                           preferred_element_type=jnp.float32) + b2_ref[...]


_head = pl.pallas_call(
    _head_body,
    grid=(NB,),
    in_specs=[
        pl.BlockSpec((BLK, D), lambda i: (i, 0)),
        pl.BlockSpec((D, FC1), lambda i: (0, 0)),
        pl.BlockSpec((1, FC1), lambda i: (0, 0)),
        pl.BlockSpec((FC1, TASKS), lambda i: (0, 0)),
        pl.BlockSpec((1, TASKS), lambda i: (0, 0)),
    ],
    out_specs=pl.BlockSpec((BLK, TASKS), lambda i: (i, 0)),
    out_shape=jax.ShapeDtypeStruct((ND, TASKS), jnp.float32),
)


def kernel(x, edge_attr, params, edge_index, batch):
    src = edge_index[0]
    dst = edge_index[1]
    batch2 = batch.reshape(ND, 1)
    zeros_nd = jnp.zeros((ND, D), jnp.float32)
    zeros_pad = jnp.zeros((NDP, D), jnp.float32)
    vn0_row = params["vn_emb0"].reshape(1, D)

    h0, hcur, onehot = _prep(x, params["node_enc_w"],
                             params["node_enc_b"].reshape(1, D),
                             vn0_row, batch2)

    es = [_edge_enc(edge_attr, params["edge_w"][l],
                    params["edge_b"][l].reshape(1, D)) for l in range(NL)]

    vn = jnp.broadcast_to(vn0_row, (NG, D))
    jk = zeros_nd
    hprev = h0
    for l in range(NL):
        agg2 = _sc_msg(hcur, es[l], src, dst, zeros_pad)
        scale = jnp.broadcast_to((1.0 + params["eps"][l]).reshape(1, 1),
                                 (1, D))
        z1, stats = _mlp(hcur, agg2, scale,
                         params["mlp_w1"][l], params["mlp_b1"][l].reshape(1, D2),
                         params["mlp_w2"][l], params["mlp_b2"][l].reshape(1, D))
        gamma = params["bn_gamma"][l].reshape(1, D)
        beta = params["bn_beta"][l].reshape(1, D)
        if l < NL - 1:
            z, jk, vn_new = _post_mid(
                z1, stats, hprev, jk, onehot, gamma, beta, vn,
                params["vn_w1"][l], params["vn_b1"][l].reshape(1, D2),
                params["vn_w2"][l], params["vn_b2"][l].reshape(1, D))
            hcur = _next_hcur(z, onehot, vn_new)
            vn = vn_new
        else:
            z, jk, _ = _post_last(
                z1, stats, hprev, jk, onehot, gamma, beta, vn,
                params["vn_w1"][0], params["vn_b1"][0].reshape(1, D2),
                params["vn_w2"][0], params["vn_b2"][0].reshape(1, D))
        hprev = z

    out = _head(jk, params["fc1_w"], params["fc1_b"].reshape(1, FC1),
                params["fc2_w"], params["fc2_b"].reshape(1, TASKS))
    return out
